# Initial kernel scaffold; baseline (speedup 1.0000x reference)
import jax
import jax.numpy as jnp
from jax.experimental import pallas as pl


def kernel(x, edge_index):
    raise NotImplementedError("write your pallas kernel here")


# speedup vs baseline: 4.0303x; 4.0303x over previous
import functools

import jax
import jax.numpy as jnp
from jax import lax
from jax.experimental import pallas as pl
from jax.experimental.pallas import tpu as pltpu
from jax.experimental.pallas import tpu_sc as plsc

N_NODES = 10000
D = 128
E = 320000
NUM_CORES = 2
NUM_SUBCORES = 16
NW = NUM_CORES * NUM_SUBCORES
E_PER_W = E // NW
CHUNK = 80
NCHUNK = E_PER_W // CHUNK


def _normalize_body(x_ref, h_ref):
    xv = x_ref[...]
    ss = jnp.sum(xv * xv, axis=-1, keepdims=True)
    nrm = jnp.sqrt(ss)
    h_ref[...] = xv / jnp.maximum(nrm, 1e-12)


def _normalize(x):
    return pl.pallas_call(
        _normalize_body,
        out_shape=jax.ShapeDtypeStruct((N_NODES, D), jnp.float32),
    )(x)


def _sc_edge_dot(h, src, dst):
    mesh = plsc.VectorSubcoreMesh(core_axis_name="c", subcore_axis_name="s")

    @functools.partial(
        pl.kernel,
        mesh=mesh,
        compiler_params=pltpu.CompilerParams(needs_layout_passes=False),
        out_type=jax.ShapeDtypeStruct((E,), jnp.float32),
        scratch_types=[
            pltpu.VMEM((E_PER_W,), jnp.int32),
            pltpu.VMEM((E_PER_W,), jnp.int32),
            pltpu.VMEM((CHUNK, D), jnp.float32),
            pltpu.VMEM((CHUNK, D), jnp.float32),
            pltpu.VMEM((CHUNK, D), jnp.float32),
            pltpu.VMEM((CHUNK, D), jnp.float32),
            pltpu.VMEM((E_PER_W,), jnp.float32),
            pltpu.SemaphoreType.DMA,
            pltpu.SemaphoreType.DMA,
            pltpu.SemaphoreType.DMA,
            pltpu.SemaphoreType.DMA,
        ],
    )
    def k(h_hbm, src_hbm, dst_hbm, out_hbm,
          src_v, dst_v, bs0, bs1, bd0, bd1, out_v,
          sem_s0, sem_s1, sem_d0, sem_d1):
        wid = lax.axis_index("s") * NUM_CORES + lax.axis_index("c")
        base = wid * E_PER_W
        pltpu.sync_copy(src_hbm.at[pl.ds(base, E_PER_W)], src_v)
        pltpu.sync_copy(dst_hbm.at[pl.ds(base, E_PER_W)], dst_v)

        bufs = ((bs0, bd0, sem_s0, sem_d0), (bs1, bd1, sem_s1, sem_d1))

        def start(c, b):
            bs, bd, ss, sd = bufs[b]
            pltpu.async_copy(h_hbm.at[src_v.at[pl.ds(c * CHUNK, CHUNK)]], bs, ss)
            pltpu.async_copy(h_hbm.at[dst_v.at[pl.ds(c * CHUNK, CHUNK)]], bd, sd)

        def wait(c, b):
            bs, bd, ss, sd = bufs[b]
            pltpu.make_async_copy(h_hbm.at[src_v.at[pl.ds(c * CHUNK, CHUNK)]], bs, ss).wait()
            pltpu.make_async_copy(h_hbm.at[dst_v.at[pl.ds(c * CHUNK, CHUNK)]], bd, sd).wait()

        lane = lax.iota(jnp.int32, 16)

        def compute(c, b):
            bs, bd, _, _ = bufs[b]

            def group_body(gi, _):
                eb = gi * 16
                res = jnp.zeros((16,), jnp.float32)
                for e in range(16):
                    acc = bs[eb + e, pl.ds(0, 16)] * bd[eb + e, pl.ds(0, 16)]
                    for g in range(1, 8):
                        acc = acc + bs[eb + e, pl.ds(g * 16, 16)] * bd[eb + e, pl.ds(g * 16, 16)]
                    res = jnp.where(lane == e, jnp.sum(acc), res)
                out_v[pl.ds(c * CHUNK + eb, 16)] = res
                return 0

            lax.fori_loop(0, CHUNK // 16, group_body, 0)

        start(0, 0)
        start(1, 1)

        def pair_body(g, _):
            for b in range(2):
                c = 2 * g + b
                wait(c, b)
                compute(c, b)

                @pl.when(c + 2 < NCHUNK)
                def _():
                    start(c + 2, b)
            return 0

        lax.fori_loop(0, NCHUNK // 2, pair_body, 0)
        wait(NCHUNK - 1, 0)
        compute(NCHUNK - 1, 0)

        pltpu.sync_copy(out_v, out_hbm.at[pl.ds(base, E_PER_W)])

    return k(h, src, dst)


def kernel(x, edge_index):
    h = _normalize(x)
    ei = edge_index.astype(jnp.int32)
    score = _sc_edge_dot(h, ei[0], ei[1])
    return score.reshape(E, 1)


# speedup vs baseline: 4.6810x; 1.1614x over previous
import functools

import jax
import jax.numpy as jnp
from jax import lax
from jax.experimental import pallas as pl
from jax.experimental.pallas import tpu as pltpu
from jax.experimental.pallas import tpu_sc as plsc

N_NODES = 10000
D = 128
E = 320000
NUM_CORES = 2
NUM_SUBCORES = 16
NW = NUM_CORES * NUM_SUBCORES
E_PER_W = E // NW
CHUNK = 80
NCHUNK = E_PER_W // CHUNK


def _normalize_body(x_ref, h_ref):
    xv = x_ref[...]
    ss = jnp.sum(xv * xv, axis=-1, keepdims=True)
    nrm = jnp.sqrt(ss)
    h_ref[...] = xv / jnp.maximum(nrm, 1e-12)


def _normalize(x):
    return pl.pallas_call(
        _normalize_body,
        out_shape=jax.ShapeDtypeStruct((N_NODES, D), jnp.float32),
    )(x)


def _sc_edge_dot(h, src, dst):
    mesh = plsc.VectorSubcoreMesh(core_axis_name="c", subcore_axis_name="s")

    @functools.partial(
        pl.kernel,
        mesh=mesh,
        compiler_params=pltpu.CompilerParams(needs_layout_passes=False),
        out_type=jax.ShapeDtypeStruct((E,), jnp.float32),
        scratch_types=[
            pltpu.VMEM((E_PER_W,), jnp.int32),
            pltpu.VMEM((E_PER_W,), jnp.int32),
            pltpu.VMEM((CHUNK, D), jnp.float32),
            pltpu.VMEM((CHUNK, D), jnp.float32),
            pltpu.VMEM((CHUNK, D), jnp.float32),
            pltpu.VMEM((CHUNK, D), jnp.float32),
            pltpu.VMEM((E_PER_W,), jnp.float32),
            pltpu.SemaphoreType.DMA,
            pltpu.SemaphoreType.DMA,
            pltpu.SemaphoreType.DMA,
            pltpu.SemaphoreType.DMA,
        ],
    )
    def k(h_hbm, src_hbm, dst_hbm, out_hbm,
          src_v, dst_v, bs0, bs1, bd0, bd1, out_v,
          sem_s0, sem_s1, sem_d0, sem_d1):
        wid = lax.axis_index("s") * NUM_CORES + lax.axis_index("c")
        base = wid * E_PER_W
        pltpu.sync_copy(src_hbm.at[pl.ds(base, E_PER_W)], src_v)
        pltpu.sync_copy(dst_hbm.at[pl.ds(base, E_PER_W)], dst_v)

        bufs = ((bs0, bd0, sem_s0, sem_d0), (bs1, bd1, sem_s1, sem_d1))

        def start(c, b):
            bs, bd, ss, sd = bufs[b]
            pltpu.async_copy(h_hbm.at[src_v.at[pl.ds(c * CHUNK, CHUNK)]], bs, ss)
            pltpu.async_copy(h_hbm.at[dst_v.at[pl.ds(c * CHUNK, CHUNK)]], bd, sd)

        def wait(c, b):
            bs, bd, ss, sd = bufs[b]
            pltpu.make_async_copy(h_hbm.at[src_v.at[pl.ds(c * CHUNK, CHUNK)]], bs, ss).wait()
            pltpu.make_async_copy(h_hbm.at[dst_v.at[pl.ds(c * CHUNK, CHUNK)]], bd, sd).wait()

        lane = lax.iota(jnp.int32, 16)
        bitrev4 = (0, 8, 4, 12, 2, 10, 6, 14, 1, 9, 5, 13, 3, 11, 7, 15)
        perms = {k: (lane ^ k).astype(jnp.int32) for k in (8, 4, 2, 1)}
        masks = {k: (lane & k) == 0 for k in (8, 4, 2, 1)}

        def fold(v, k):
            return v + jnp.take_along_axis(v, perms[k], axis=0)

        def compute(c, b):
            bs, bd, _, _ = bufs[b]

            def group_body(gi, _):
                eb = gi * 16
                vecs = []
                for i in range(16):
                    e = eb + bitrev4[i]
                    prod = [bs[e, pl.ds(g * 16, 16)] * bd[e, pl.ds(g * 16, 16)]
                            for g in range(8)]
                    t = [prod[2 * j] + prod[2 * j + 1] for j in range(4)]
                    u = [t[0] + t[1], t[2] + t[3]]
                    vecs.append(u[0] + u[1])
                for k in (8, 4, 2, 1):
                    vecs = [jnp.where(masks[k], fold(vecs[2 * j], k),
                                      fold(vecs[2 * j + 1], k))
                            for j in range(len(vecs) // 2)]
                out_v[pl.ds(c * CHUNK + eb, 16)] = vecs[0]
                return 0

            lax.fori_loop(0, CHUNK // 16, group_body, 0)

        start(0, 0)
        start(1, 1)

        def pair_body(g, _):
            for b in range(2):
                c = 2 * g + b
                wait(c, b)
                compute(c, b)

                @pl.when(c + 2 < NCHUNK)
                def _():
                    start(c + 2, b)
            return 0

        lax.fori_loop(0, NCHUNK // 2, pair_body, 0)
        wait(NCHUNK - 1, 0)
        compute(NCHUNK - 1, 0)

        pltpu.sync_copy(out_v, out_hbm.at[pl.ds(base, E_PER_W)])

    return k(h, src, dst)


def kernel(x, edge_index):
    h = _normalize(x)
    ei = edge_index.astype(jnp.int32)
    score = _sc_edge_dot(h, ei[0], ei[1])
    return score.reshape(E, 1)
